# baseline (speedup 1.0000x reference)
import functools

import jax
import jax.numpy as jnp
from jax import lax
from jax.experimental import pallas as pl
from jax.experimental.pallas import tpu as pltpu
from jax.experimental.pallas import tpu_sc as plsc

_NC = 2
_NS = 16
_CH = 128
_IB = 40
_ZR = 128
_GE = 512


def _make_seg_sum(n_out_pad, table_rows, src_rows, per_core_rows,
                  rows_per_tile, src_core_stride):
  n_blocks = rows_per_tile // _IB
  nacc = n_out_pad
  zspan = nacc // _NS
  zfills = zspan // _ZR
  zrem = zspan - zfills * _ZR
  out_rows = n_out_pad // _NS
  mesh = plsc.VectorSubcoreMesh(core_axis_name="c", subcore_axis_name="s")

  @functools.partial(
      pl.kernel,
      out_type=jax.ShapeDtypeStruct((_NC, n_out_pad, 16), jnp.float32),
      mesh=mesh,
      compiler_params=pltpu.CompilerParams(use_tc_tiling_on_sc=False),
      scratch_types=[
          pltpu.VMEM((_ZR, 16), jnp.float32),
          pltpu.VMEM((_IB * _CH,), jnp.int32),
          pltpu.VMEM((_IB * _CH,), jnp.int32),
          pltpu.VMEM((_GE, 16), jnp.float32),
          pltpu.VMEM((_GE, 16), jnp.float32),
          pltpu.VMEM_SHARED((nacc, 16), jnp.float32),
          pltpu.SemaphoreType.DMA((2,)),
          pltpu.SemaphoreType.DMA((2,)),
          pltpu.SemaphoreType.DMA,
      ],
  )
  def seg(table_hbm, src_hbm, dst_hbm, out_hbm, zbuf, sbuf, dbuf, slab0,
          slab1, acc, gsem, ssem, zsem):
    c = lax.axis_index("c")
    s = lax.axis_index("s")
    slabs = (slab0, slab1)

    def zfill(i, carry):
      zbuf[i] = jnp.zeros((16,), jnp.float32)
      return carry
    lax.fori_loop(0, _ZR, zfill, 0)

    zbase = s * zspan

    def zcopy(k, carry):
      for b in range(8):
        pltpu.async_copy(
            zbuf, acc.at[pl.ds(zbase + (k * 8 + b) * _ZR, _ZR)], zsem)
      for b in range(8):
        pltpu.make_async_copy(
            zbuf, acc.at[pl.ds(zbase + (k * 8 + b) * _ZR, _ZR)], zsem).wait()
      return carry
    lax.fori_loop(0, zfills // 8, zcopy, 0)
    for k in range(zfills - (zfills // 8) * 8):
      pltpu.sync_copy(zbuf,
                      acc.at[pl.ds(zbase + ((zfills // 8) * 8 + k) * _ZR,
                                   _ZR)])
    if zrem:
      pltpu.sync_copy(zbuf.at[pl.ds(0, zrem)],
                      acc.at[pl.ds(zbase + zfills * _ZR, zrem)])
    plsc.subcore_barrier()

    dst_cb = (c * (per_core_rows // _IB)
              + s * (rows_per_tile // _IB))
    src_cb = c * (src_core_stride // _IB) + dst_cb

    ngrp = (_IB * _CH) // _GE

    def sidx(t):
      return sbuf.at[pl.ds(t * _GE, _GE)]

    def didx(t):
      return dbuf.at[pl.ds(t * _GE, _GE)]

    def chunk(ci, carry):
      pltpu.sync_copy(src_hbm.at[src_cb + ci], sbuf)
      pltpu.sync_copy(dst_hbm.at[dst_cb + ci], dbuf)

      def tpair(tp, carry2):
        for tt in range(2):
          t = tp * 2 + tt
          sl = slabs[tt]

          @pl.when(tp > 0)
          def _wait_scatter(sl=sl, t=t):
            pltpu.make_async_copy(
                sl, acc.at[didx(t - 2)], ssem.at[tt]).wait()
          pltpu.async_copy(table_hbm.at[sidx(t)], sl, gsem.at[tt])

          po = 1 - tt
          psl = slabs[po]

          def _scat(psl=psl, po=po, t=t):
            pltpu.make_async_copy(
                table_hbm.at[sidx(t - 1)], psl, gsem.at[po]).wait()
            pltpu.async_copy(
                psl, acc.at[didx(t - 1)], ssem.at[po], add=True)
          if tt == 0:
            pl.when(tp > 0)(_scat)
          else:
            _scat()
        return carry2
      lax.fori_loop(0, ngrp // 2, tpair, 0)

      pltpu.make_async_copy(
          table_hbm.at[sidx(ngrp - 1)], slabs[1], gsem.at[1]).wait()
      pltpu.async_copy(slabs[1], acc.at[didx(ngrp - 1)], ssem.at[1], add=True)
      pltpu.make_async_copy(slabs[0], acc.at[didx(ngrp - 2)], ssem.at[0]).wait()
      pltpu.make_async_copy(slabs[1], acc.at[didx(ngrp - 1)], ssem.at[1]).wait()
      return carry
    lax.fori_loop(0, n_blocks, chunk, 0)
    plsc.subcore_barrier()

    ob = s * out_rows
    pltpu.sync_copy(acc.at[pl.ds(ob, out_rows)],
                    out_hbm.at[c, pl.ds(ob, out_rows)])

  return seg


def _dense1(agg1, x, W1_l, W1_r, b1, W2_l, W2_r, b2):
  n = x.shape[0]
  r = 2000
  grid = (n // r,)

  def body(a_ref, x_ref, w1l_ref, w1r_ref, b1_ref, w2l_ref, w2r_ref, b2_ref,
           hw_ref, hr_ref, inv_ref):
    a = a_ref[0] + a_ref[1]
    inv = 1.0 / jnp.maximum(a[:, 11:12], 1.0)
    mean1 = a[:, :11] * inv
    h = jnp.maximum(
        jnp.dot(mean1, w1l_ref[...], preferred_element_type=jnp.float32)
        + jnp.dot(x_ref[...], w1r_ref[...], preferred_element_type=jnp.float32)
        + b1_ref[...], 0.0)
    hl = jnp.dot(h, w2l_ref[...], preferred_element_type=jnp.float32)
    hw_ref[0] = hl[:, :16]
    hw_ref[1] = hl[:, 16:]
    hr_ref[...] = (jnp.dot(h, w2r_ref[...], preferred_element_type=jnp.float32)
                   + b2_ref[...])
    inv_ref[...] = inv

  return pl.pallas_call(
      body,
      grid=grid,
      in_specs=[
          pl.BlockSpec((2, r, 16), lambda i: (0, i, 0)),
          pl.BlockSpec((r, 11), lambda i: (i, 0)),
          pl.BlockSpec((11, 64), lambda i: (0, 0)),
          pl.BlockSpec((11, 64), lambda i: (0, 0)),
          pl.BlockSpec((1, 64), lambda i: (0, 0)),
          pl.BlockSpec((64, 32), lambda i: (0, 0)),
          pl.BlockSpec((64, 32), lambda i: (0, 0)),
          pl.BlockSpec((1, 32), lambda i: (0, 0)),
      ],
      out_specs=[
          pl.BlockSpec((2, r, 16), lambda i: (0, i, 0)),
          pl.BlockSpec((r, 32), lambda i: (i, 0)),
          pl.BlockSpec((r, 1), lambda i: (i, 0)),
      ],
      out_shape=[
          jax.ShapeDtypeStruct((2, n, 16), jnp.float32),
          jax.ShapeDtypeStruct((n, 32), jnp.float32),
          jax.ShapeDtypeStruct((n, 1), jnp.float32),
      ],
  )(agg1, x, W1_l, W1_r, b1.reshape(1, -1), W2_l, W2_r, b2.reshape(1, -1))


def _dense2(agg2, inv, hr, Wc, bc):
  n = hr.shape[0]
  r = 2000
  grid = (n // r,)

  def body(a_ref, inv_ref, hr_ref, wc_ref, bc_ref, logits_ref, emb_ref):
    a = jnp.concatenate([a_ref[0], a_ref[1]], axis=1)
    emb = jnp.maximum(a * inv_ref[...] + hr_ref[...], 0.0)
    emb_ref[...] = emb
    logits_ref[...] = (jnp.dot(emb, wc_ref[...],
                               preferred_element_type=jnp.float32)
                       + bc_ref[...])

  return pl.pallas_call(
      body,
      grid=grid,
      in_specs=[
          pl.BlockSpec((2, r, 16), lambda i: (0, i, 0)),
          pl.BlockSpec((r, 1), lambda i: (i, 0)),
          pl.BlockSpec((r, 32), lambda i: (i, 0)),
          pl.BlockSpec((32, 3), lambda i: (0, 0)),
          pl.BlockSpec((1, 3), lambda i: (0, 0)),
      ],
      out_specs=[
          pl.BlockSpec((r, 3), lambda i: (i, 0)),
          pl.BlockSpec((r, 32), lambda i: (i, 0)),
      ],
      out_shape=[
          jax.ShapeDtypeStruct((n, 3), jnp.float32),
          jax.ShapeDtypeStruct((n, 32), jnp.float32),
      ],
  )(agg2, inv, hr, Wc, bc.reshape(1, -1))


def kernel(x, edge_index, W1_l, W1_r, b1, W2_l, W2_r, b2, Wc, bc):
  n = x.shape[0]
  e = edge_index.shape[1]

  row_quant = _CH * _NC * _NS * _IB
  ep = ((e + row_quant - 1) // row_quant) * row_quant
  rows = ep // _CH
  src = edge_index[0].astype(jnp.int32)
  dst = edge_index[1].astype(jnp.int32)
  src_p = jnp.concatenate(
      [src, jnp.zeros((ep - e,), jnp.int32)]).reshape(-1, _IB * _CH)
  dst_p = jnp.concatenate(
      [dst, jnp.full((ep - e,), n, jnp.int32)]).reshape(-1, _IB * _CH)

  np8 = 8 * _NS
  n_out_pad = ((n + np8) // np8) * np8

  xp = jnp.concatenate(
      [x, jnp.ones((n, 1), x.dtype), jnp.zeros((n, 4), x.dtype)], axis=1)
  seg1 = _make_seg_sum(
      n_out_pad=n_out_pad, table_rows=n, src_rows=rows,
      per_core_rows=rows // _NC, rows_per_tile=rows // (_NC * _NS),
      src_core_stride=0)
  agg1 = seg1(xp, src_p, dst_p)

  hw, hr, inv = _dense1(agg1, x, W1_l, W1_r, b1, W2_l, W2_r, b2)

  table2 = hw.reshape(2 * n, 16)
  src2 = jnp.concatenate([src_p, src_p + n], axis=0)
  seg2 = _make_seg_sum(
      n_out_pad=n_out_pad, table_rows=2 * n, src_rows=2 * rows,
      per_core_rows=0, rows_per_tile=rows // _NS,
      src_core_stride=rows)
  agg2 = seg2(table2, src2, dst_p)

  logits, emb = _dense2(agg2, inv, hr, Wc, bc)
  return (logits, emb)


# speedup vs baseline: 1.1154x; 1.1154x over previous
import functools

import jax
import jax.numpy as jnp
from jax import lax
from jax.experimental import pallas as pl
from jax.experimental.pallas import tpu as pltpu
from jax.experimental.pallas import tpu_sc as plsc

_NC = 2
_NS = 16
_CH = 128
_ZR = 128
_GE = 256
_OPC = 10
_CHK = _GE * _OPC
_RD = 4


def _make_seg_sum(n_out_pad, per_core_chunks, chunks_per_tile,
                  src_core_stride_chunks):
  nacc = n_out_pad
  zspan = nacc // _NS
  zfills = zspan // _ZR
  zrem = zspan - zfills * _ZR
  out_rows = n_out_pad // _NS
  n_pairs = chunks_per_tile // 2
  total_ops = chunks_per_tile * _OPC
  assert total_ops % _RD == 0
  mesh = plsc.VectorSubcoreMesh(core_axis_name="c", subcore_axis_name="s")

  @functools.partial(
      pl.kernel,
      out_type=jax.ShapeDtypeStruct((_NC, n_out_pad, 16), jnp.float32),
      mesh=mesh,
      compiler_params=pltpu.CompilerParams(use_tc_tiling_on_sc=False),
      scratch_types=[
          pltpu.VMEM((_ZR, 16), jnp.float32),
          pltpu.VMEM((_CHK,), jnp.int32),
          pltpu.VMEM((_CHK,), jnp.int32),
          pltpu.VMEM((_CHK,), jnp.int32),
          pltpu.VMEM((_CHK,), jnp.int32),
          pltpu.VMEM((_RD * _GE, 16), jnp.float32),
          pltpu.VMEM_SHARED((nacc, 16), jnp.float32),
          pltpu.SemaphoreType.DMA((_RD,)),
          pltpu.SemaphoreType.DMA((_RD,)),
          pltpu.SemaphoreType.DMA,
          pltpu.SemaphoreType.DMA,
      ],
  )
  def seg(table_hbm, src_hbm, dst_hbm, out_hbm, zbuf, sbuf0, sbuf1, dbuf0,
          dbuf1, slab, acc, gsem, ssem, zsem, isem):
    c = lax.axis_index("c")
    s = lax.axis_index("s")
    sbufs = (sbuf0, sbuf1)
    dbufs = (dbuf0, dbuf1)

    def slot(k):
      return slab.at[pl.ds((k % _RD) * _GE, _GE)]

    def sidx(k_rel):
      return sbufs[(k_rel // _OPC) % 2].at[pl.ds((k_rel % _OPC) * _GE, _GE)]

    def didx(k_rel):
      return dbufs[(k_rel // _OPC) % 2].at[pl.ds((k_rel % _OPC) * _GE, _GE)]

    def zfill(i, carry):
      zbuf[i] = jnp.zeros((16,), jnp.float32)
      return carry
    lax.fori_loop(0, _ZR, zfill, 0)

    zbase = s * zspan

    def zcopy(k, carry):
      for b in range(8):
        pltpu.async_copy(
            zbuf, acc.at[pl.ds(zbase + (k * 8 + b) * _ZR, _ZR)], zsem)
      for b in range(8):
        pltpu.make_async_copy(
            zbuf, acc.at[pl.ds(zbase + (k * 8 + b) * _ZR, _ZR)], zsem).wait()
      return carry
    lax.fori_loop(0, zfills // 8, zcopy, 0)
    for k in range(zfills - (zfills // 8) * 8):
      pltpu.sync_copy(zbuf,
                      acc.at[pl.ds(zbase + ((zfills // 8) * 8 + k) * _ZR,
                                   _ZR)])
    if zrem:
      pltpu.sync_copy(zbuf.at[pl.ds(0, zrem)],
                      acc.at[pl.ds(zbase + zfills * _ZR, zrem)])
    plsc.subcore_barrier()

    dst_cb = c * per_core_chunks + s * chunks_per_tile
    src_cb = c * src_core_stride_chunks + dst_cb

    pltpu.sync_copy(src_hbm.at[src_cb], sbuf0)
    pltpu.sync_copy(dst_hbm.at[dst_cb], dbuf0)

    def pair(pairi, carry):
      for cc in range(2):
        ci = pairi * 2 + cc
        for t in range(_OPC):
          k = cc * _OPC + t

          if t == 0:
            def _iwait(cc=cc, ci=ci):
              pltpu.make_async_copy(
                  src_hbm.at[src_cb + ci], sbufs[cc % 2], isem).wait()
              pltpu.make_async_copy(
                  dst_hbm.at[dst_cb + ci], dbufs[cc % 2], isem).wait()
            if cc == 0:
              pl.when(pairi > 0)(_iwait)
            else:
              _iwait()

          def _swait(k=k):
            pltpu.make_async_copy(
                slot(k), acc.at[didx(k % _OPC)], ssem.at[k % _RD]).wait()
          if k >= _RD:
            _swait()
          else:
            pl.when(pairi > 0)(_swait)
          pltpu.async_copy(table_hbm.at[sidx(k)], slot(k), gsem.at[k % _RD])

          if t == 3:
            def _ipf(cc=cc, ci=ci):
              pltpu.async_copy(
                  src_hbm.at[src_cb + ci + 1], sbufs[(cc + 1) % 2], isem)
              pltpu.async_copy(
                  dst_hbm.at[dst_cb + ci + 1], dbufs[(cc + 1) % 2], isem)
            if cc == 0:
              _ipf()
            else:
              pl.when(pairi < n_pairs - 1)(_ipf)

          def _scat(k=k):
            pltpu.make_async_copy(
                table_hbm.at[sidx(k - 3)], slot(k - 3),
                gsem.at[(k - 3) % _RD]).wait()
            pltpu.async_copy(
                slot(k - 3), acc.at[didx(k - 3)], ssem.at[(k - 3) % _RD],
                add=True)
          if k >= 3:
            _scat()
          else:
            pl.when(pairi > 0)(_scat)
      return carry
    lax.fori_loop(0, n_pairs, pair, 0)

    for i in range(3):
      k = 2 * _OPC - 3 + i
      pltpu.make_async_copy(
          table_hbm.at[sidx(k)], slot(k), gsem.at[k % _RD]).wait()
      pltpu.async_copy(slot(k), acc.at[didx(k)], ssem.at[k % _RD], add=True)
    for i in range(4):
      k = 2 * _OPC - 4 + i
      pltpu.make_async_copy(
          slot(k), acc.at[didx(k)], ssem.at[k % _RD]).wait()
    plsc.subcore_barrier()

    ob = s * out_rows
    pltpu.sync_copy(acc.at[pl.ds(ob, out_rows)],
                    out_hbm.at[c, pl.ds(ob, out_rows)])

  return seg


def _dense1(agg1, x, W1_l, W1_r, b1, W2_l, W2_r, b2):
  n = x.shape[0]
  r = 2000
  grid = (n // r,)

  def body(a_ref, x_ref, w1l_ref, w1r_ref, b1_ref, w2l_ref, w2r_ref, b2_ref,
           hw_ref, hr_ref, inv_ref):
    a = a_ref[0] + a_ref[1]
    inv = 1.0 / jnp.maximum(a[:, 11:12], 1.0)
    mean1 = a[:, :11] * inv
    h = jnp.maximum(
        jnp.dot(mean1, w1l_ref[...], preferred_element_type=jnp.float32)
        + jnp.dot(x_ref[...], w1r_ref[...], preferred_element_type=jnp.float32)
        + b1_ref[...], 0.0)
    hl = jnp.dot(h, w2l_ref[...], preferred_element_type=jnp.float32)
    hw_ref[0] = hl[:, :16]
    hw_ref[1] = hl[:, 16:]
    hr_ref[...] = (jnp.dot(h, w2r_ref[...], preferred_element_type=jnp.float32)
                   + b2_ref[...])
    inv_ref[...] = inv

  return pl.pallas_call(
      body,
      grid=grid,
      in_specs=[
          pl.BlockSpec((2, r, 16), lambda i: (0, i, 0)),
          pl.BlockSpec((r, 11), lambda i: (i, 0)),
          pl.BlockSpec((11, 64), lambda i: (0, 0)),
          pl.BlockSpec((11, 64), lambda i: (0, 0)),
          pl.BlockSpec((1, 64), lambda i: (0, 0)),
          pl.BlockSpec((64, 32), lambda i: (0, 0)),
          pl.BlockSpec((64, 32), lambda i: (0, 0)),
          pl.BlockSpec((1, 32), lambda i: (0, 0)),
      ],
      out_specs=[
          pl.BlockSpec((2, r, 16), lambda i: (0, i, 0)),
          pl.BlockSpec((r, 32), lambda i: (i, 0)),
          pl.BlockSpec((r, 1), lambda i: (i, 0)),
      ],
      out_shape=[
          jax.ShapeDtypeStruct((2, n, 16), jnp.float32),
          jax.ShapeDtypeStruct((n, 32), jnp.float32),
          jax.ShapeDtypeStruct((n, 1), jnp.float32),
      ],
  )(agg1, x, W1_l, W1_r, b1.reshape(1, -1), W2_l, W2_r, b2.reshape(1, -1))


def _dense2(agg2, inv, hr, Wc, bc):
  n = hr.shape[0]
  r = 2000
  grid = (n // r,)

  def body(a_ref, inv_ref, hr_ref, wc_ref, bc_ref, logits_ref, emb_ref):
    a = jnp.concatenate([a_ref[0], a_ref[1]], axis=1)
    emb = jnp.maximum(a * inv_ref[...] + hr_ref[...], 0.0)
    emb_ref[...] = emb
    logits_ref[...] = (jnp.dot(emb, wc_ref[...],
                               preferred_element_type=jnp.float32)
                       + bc_ref[...])

  return pl.pallas_call(
      body,
      grid=grid,
      in_specs=[
          pl.BlockSpec((2, r, 16), lambda i: (0, i, 0)),
          pl.BlockSpec((r, 1), lambda i: (i, 0)),
          pl.BlockSpec((r, 32), lambda i: (i, 0)),
          pl.BlockSpec((32, 3), lambda i: (0, 0)),
          pl.BlockSpec((1, 3), lambda i: (0, 0)),
      ],
      out_specs=[
          pl.BlockSpec((r, 3), lambda i: (i, 0)),
          pl.BlockSpec((r, 32), lambda i: (i, 0)),
      ],
      out_shape=[
          jax.ShapeDtypeStruct((n, 3), jnp.float32),
          jax.ShapeDtypeStruct((n, 32), jnp.float32),
      ],
  )(agg2, inv, hr, Wc, bc.reshape(1, -1))


def kernel(x, edge_index, W1_l, W1_r, b1, W2_l, W2_r, b2, Wc, bc):
  n = x.shape[0]
  e = edge_index.shape[1]

  quant = _CHK * _NC * _NS * 2
  ep = ((e + quant - 1) // quant) * quant
  chunks = ep // _CHK
  src = edge_index[0].astype(jnp.int32)
  dst = edge_index[1].astype(jnp.int32)
  src_p = jnp.concatenate(
      [src, jnp.zeros((ep - e,), jnp.int32)]).reshape(-1, _CHK)
  dst_p = jnp.concatenate(
      [dst, jnp.full((ep - e,), n, jnp.int32)]).reshape(-1, _CHK)

  np8 = 8 * _NS
  n_out_pad = ((n + np8) // np8) * np8

  xp = jnp.concatenate(
      [x, jnp.ones((n, 1), x.dtype), jnp.zeros((n, 4), x.dtype)], axis=1)
  seg1 = _make_seg_sum(
      n_out_pad=n_out_pad, per_core_chunks=chunks // _NC,
      chunks_per_tile=chunks // (_NC * _NS), src_core_stride_chunks=0)
  agg1 = seg1(xp, src_p, dst_p)

  hw, hr, inv = _dense1(agg1, x, W1_l, W1_r, b1, W2_l, W2_r, b2)

  table2 = hw.reshape(2 * n, 16)
  src2 = jnp.concatenate([src_p, src_p + n], axis=0)
  seg2 = _make_seg_sum(
      n_out_pad=n_out_pad, per_core_chunks=0,
      chunks_per_tile=chunks // _NS, src_core_stride_chunks=chunks)
  agg2 = seg2(table2, src2, dst_p)

  logits, emb = _dense2(agg2, inv, hr, Wc, bc)
  return (logits, emb)
